# Initial kernel scaffold; baseline (speedup 1.0000x reference)
import jax
import jax.numpy as jnp
from jax.experimental import pallas as pl


def kernel(x, tables):
    raise NotImplementedError("write your pallas kernel here")


# speedup vs baseline: 1.9465x; 1.9465x over previous
import functools

import jax
import jax.numpy as jnp
from jax import lax
from jax.experimental import pallas as pl
from jax.experimental.pallas import tpu as pltpu
from jax.experimental.pallas import tpu_sc as plsc

NUM_FIELDS = 26
VOCAB = 1000
D = 16
BATCH = 4096
NJ = NUM_FIELDS - 1

N_ROWS = NUM_FIELDS * BATCH * NJ
NW = 32
ROWS_PER_W = N_ROWS // NW
IDX_W = 128
G = 13
ROWS_PER_G = IDX_W * G
GROUPS = ROWS_PER_W // ROWS_PER_G


def _make_kernel():
    mesh = plsc.VectorSubcoreMesh(core_axis_name="c", subcore_axis_name="s")

    @functools.partial(
        pl.kernel,
        mesh=mesh,
        out_type=jax.ShapeDtypeStruct((N_ROWS, D), jnp.float32),
        compiler_params=pltpu.CompilerParams(use_tc_tiling_on_sc=False),
        scratch_types=[
            pltpu.VMEM((ROWS_PER_G,), jnp.int32),
            pltpu.VMEM((ROWS_PER_G, D), jnp.float32),
            pltpu.SemaphoreType.DMA,
        ],
    )
    def k(tab_hbm, idx_hbm, out_hbm, idx_v, rows_v, sem):
        nc = 2
        wid = lax.axis_index("s") * nc + lax.axis_index("c")
        row0 = wid * ROWS_PER_W

        def body(g, carry):
            base = row0 + g * ROWS_PER_G
            pltpu.sync_copy(idx_hbm.at[pl.ds(base, ROWS_PER_G)], idx_v)
            copies = []
            for t in range(G):
                copies.append(
                    pltpu.async_copy(
                        tab_hbm.at[idx_v.at[pl.ds(t * IDX_W, IDX_W)]],
                        rows_v.at[pl.ds(t * IDX_W, IDX_W)],
                        sem,
                    )
                )
            for cpy in copies:
                cpy.wait()
            pltpu.sync_copy(
                rows_v, out_hbm.at[pl.ds(row0 + g * ROWS_PER_G, ROWS_PER_G)]
            )
            return carry

        lax.fori_loop(0, GROUPS, body, 0)

    return k


_GATHER = _make_kernel()


def kernel(x, tables):
    tab = tables.reshape(NUM_FIELDS * NUM_FIELDS * VOCAB, D)
    i_ar = jnp.arange(NUM_FIELDS, dtype=jnp.int32)[:, None, None]
    jj = jnp.arange(NJ, dtype=jnp.int32)[None, None, :]
    j = jj + (jj >= i_ar).astype(jnp.int32)
    base = (i_ar * NUM_FIELDS + j) * VOCAB
    idx = base + x.T[:, :, None]
    idx = idx.reshape(N_ROWS)
    out = _GATHER(tab, idx)
    return out.reshape(NUM_FIELDS, BATCH, NJ, D)


# speedup vs baseline: 5.8870x; 3.0244x over previous
import functools

import jax
import jax.numpy as jnp
from jax import lax
from jax.experimental import pallas as pl
from jax.experimental.pallas import tpu as pltpu
from jax.experimental.pallas import tpu_sc as plsc

F = 26
V = 1000
D = 16
B = 4096
NJ = F - 1

NC = 2
NS = 16
IPC = F // NC
PAIRS_PC = IPC * NJ
P1_ITERS = -(-PAIRS_PC // NS)
BPT = B // NS
CH = 128
NCH = BPT // CH


def _make_kernel():
    mesh = plsc.VectorSubcoreMesh(core_axis_name="c", subcore_axis_name="s")

    @functools.partial(
        pl.kernel,
        mesh=mesh,
        out_type=[
            jax.ShapeDtypeStruct((F, B, NJ, D), jnp.float32),
            jax.ShapeDtypeStruct((F * V, NJ, D), jnp.float32),
        ],
        compiler_params=pltpu.CompilerParams(use_tc_tiling_on_sc=False),
        scratch_types=[
            pltpu.VMEM((V, D), jnp.float32),
            pltpu.VMEM((BPT,), jnp.int32),
            pltpu.VMEM((CH, NJ, D), jnp.float32),
            pltpu.SemaphoreType.DMA,
        ],
    )
    def k(xt_hbm, tab_hbm, out_hbm, tab2_hbm, blk_v, xcol_v, rows_v, sem):
        c = lax.axis_index("c")
        s = lax.axis_index("s")

        for t in range(P1_ITERS):
            p = s + NS * t

            @pl.when(p < PAIRS_PC)
            def _():
                i_loc = p // NJ
                jj = p % NJ
                i_glob = c * IPC + i_loc
                j = jj + jnp.where(jj >= i_glob, 1, 0).astype(jj.dtype)
                pltpu.sync_copy(tab_hbm.at[i_glob, j], blk_v)
                pltpu.sync_copy(blk_v, tab2_hbm.at[pl.ds(i_glob * V, V), jj, :])

        plsc.subcore_barrier()

        b0 = s * BPT
        for i_loc in range(IPC):
            i_glob = c * IPC + i_loc
            pltpu.sync_copy(xt_hbm.at[i_glob, pl.ds(b0, BPT)], xcol_v)
            off = (i_glob * V).astype(jnp.int32)
            for q in range(BPT // 16):
                sl = pl.ds(q * 16, 16)
                xcol_v[sl] = xcol_v[sl] + off
            for kk in range(NCH):
                pltpu.async_copy(
                    tab2_hbm.at[xcol_v.at[pl.ds(kk * CH, CH)]], rows_v, sem
                ).wait()
                pltpu.sync_copy(
                    rows_v, out_hbm.at[i_glob, pl.ds(b0 + kk * CH, CH)]
                )

    return k


_GATHER = _make_kernel()


def kernel(x, tables):
    out, _ = _GATHER(x.T, tables)
    return out


# speedup vs baseline: 16.6142x; 2.8222x over previous
import functools

import jax
import jax.numpy as jnp
from jax import lax
from jax.experimental import pallas as pl
from jax.experimental.pallas import tpu as pltpu
from jax.experimental.pallas import tpu_sc as plsc

F = 26
V = 1000
D = 16
B = 4096
NJ = F - 1

NC = 2
NS = 16
NW = NC * NS
PAIRS = F * NJ
P_IT = -(-PAIRS // NW)
QV = B // 16


def _make_kernel():
    mesh = plsc.VectorSubcoreMesh(core_axis_name="c", subcore_axis_name="s")

    @functools.partial(
        pl.kernel,
        mesh=mesh,
        out_type=jax.ShapeDtypeStruct((F, NJ, D, B), jnp.float32),
        compiler_params=pltpu.CompilerParams(
            use_tc_tiling_on_sc=False, needs_layout_passes=False
        ),
        scratch_types=[
            pltpu.VMEM((D, V), jnp.float32),
            pltpu.VMEM((B,), jnp.int32),
            pltpu.VMEM((D, B), jnp.float32),
            pltpu.SemaphoreType.DMA,
        ],
    )
    def k(xt_hbm, tab_hbm, out_hbm, tbl_v, xcol_v, outc_v, sem):
        c = lax.axis_index("c")
        s = lax.axis_index("s")
        w = s * NC + c

        dsplat = [jnp.full((16,), d, dtype=jnp.int32) for d in range(D)]

        for t in range(P_IT):
            p = w + NW * t

            @pl.when(p < PAIRS)
            def _():
                i = p // NJ
                jj = p % NJ
                j = jj + jnp.where(jj >= i, 1, 0).astype(jj.dtype)
                pltpu.sync_copy(tab_hbm.at[i, j], tbl_v)
                pltpu.sync_copy(xt_hbm.at[i], xcol_v)

                def body(q, carry):
                    sl = pl.ds(q * 16, 16)
                    xv = xcol_v[sl]
                    for d in range(D):
                        outc_v[d, sl] = plsc.load_gather(
                            tbl_v, [dsplat[d], xv]
                        )
                    return carry

                lax.fori_loop(0, QV, body, 0)
                pltpu.sync_copy(outc_v, out_hbm.at[i, jj])

    return k


_GATHER = _make_kernel()


def kernel(x, tables):
    tt = tables.transpose(0, 1, 3, 2)
    out = _GATHER(x.T, tt)
    return out.transpose(0, 3, 1, 2)


# speedup vs baseline: 29.3441x; 1.7662x over previous
import functools

import jax
import jax.numpy as jnp
from jax import lax
from jax.experimental import pallas as pl
from jax.experimental.pallas import tpu as pltpu
from jax.experimental.pallas import tpu_sc as plsc

F = 26
V = 1000
D = 16
B = 4096
NJ = F - 1

NC = 2
NS = 16
NW = NC * NS
PAIRS = F * NJ
P_IT = -(-PAIRS // NW)
BH = B // 2
QH = BH // 16


def _make_kernel():
    mesh = plsc.VectorSubcoreMesh(core_axis_name="c", subcore_axis_name="s")

    @functools.partial(
        pl.kernel,
        mesh=mesh,
        out_type=jax.ShapeDtypeStruct((F, NJ, D, B), jnp.float32),
        compiler_params=pltpu.CompilerParams(
            use_tc_tiling_on_sc=False, needs_layout_passes=False
        ),
        scratch_types=[
            pltpu.VMEM((D, V), jnp.float32),
            pltpu.VMEM((B,), jnp.int32),
            pltpu.VMEM((D, BH), jnp.float32),
            pltpu.VMEM((D, BH), jnp.float32),
            pltpu.SemaphoreType.DMA,
        ],
    )
    def k(xt_hbm, tab_hbm, out_hbm, tbl_v, xcol_v, outa_v, outb_v, sem):
        c = lax.axis_index("c")
        s = lax.axis_index("s")
        w = s * NC + c

        dsplat = [jnp.full((16,), d, dtype=jnp.int32) for d in range(D)]

        def fill(out_ref, h):
            @plsc.parallel_loop(0, QH, 1, unroll=8)
            def _body(q):
                sl = pl.ds(q * 16, 16)
                xv = xcol_v[pl.ds(h * BH + q * 16, 16)]
                for d in range(D):
                    out_ref[d, sl] = plsc.load_gather(tbl_v, [dsplat[d], xv])

        def wait_half(out_ref):
            pltpu.make_async_copy(
                out_hbm.at[0, 0, :, pl.ds(0, BH)], out_ref, sem
            ).wait()

        def run_pair(p, wait_a, wait_b):
            i = p // NJ
            jj = p % NJ
            j = jj + jnp.where(jj >= i, 1, 0).astype(jj.dtype)
            pltpu.sync_copy(tab_hbm.at[i, j], tbl_v)
            pltpu.sync_copy(xt_hbm.at[i], xcol_v)
            if wait_a:
                wait_half(outa_v)
            fill(outa_v, 0)
            if wait_b:
                wait_half(outb_v)
            pltpu.async_copy(
                outa_v, out_hbm.at[i, jj, :, pl.ds(0, BH)], sem
            )
            fill(outb_v, 1)
            pltpu.async_copy(
                outb_v, out_hbm.at[i, jj, :, pl.ds(BH, BH)], sem
            )

        run_pair(w, False, False)

        def body(t, carry):
            p = w + NW * t

            @pl.when(p < PAIRS)
            def _():
                run_pair(p, True, True)

            return carry

        lax.fori_loop(1, P_IT, body, 0)
        wait_half(outa_v)
        wait_half(outb_v)

    return k


_GATHER = _make_kernel()


def kernel(x, tables):
    tt = tables.transpose(0, 1, 3, 2)
    out = _GATHER(x.T, tt)
    return out.transpose(0, 3, 1, 2)


# speedup vs baseline: 50.1622x; 1.7094x over previous
import functools

import jax
import jax.numpy as jnp
from jax import lax
from jax.experimental import pallas as pl
from jax.experimental.pallas import tpu as pltpu
from jax.experimental.pallas import tpu_sc as plsc

F = 26
V = 1000
D = 16
B = 4096
NJ = F - 1

NC = 2
NS = 16
NW = NC * NS
PAIRS = F * NJ
P_IT = -(-PAIRS // NW)
BH = B // 2
QH = BH // 16


def _make_kernel():
    mesh = plsc.VectorSubcoreMesh(core_axis_name="c", subcore_axis_name="s")

    @functools.partial(
        pl.kernel,
        mesh=mesh,
        out_type=jax.ShapeDtypeStruct((F, NJ, 2, B // 128, 8, 128), jnp.float32),
        compiler_params=pltpu.CompilerParams(
            use_tc_tiling_on_sc=False, needs_layout_passes=False
        ),
        scratch_types=[
            pltpu.VMEM((D, V), jnp.float32),
            pltpu.VMEM((B,), jnp.int32),
            pltpu.VMEM((2, BH // 128, 8, 128), jnp.float32),
            pltpu.VMEM((2, BH // 128, 8, 128), jnp.float32),
            pltpu.SemaphoreType.DMA,
        ],
    )
    def k(xt_hbm, tab_hbm, out_hbm, tbl_v, xcol_v, outa_v, outb_v, sem):
        c = lax.axis_index("c")
        s = lax.axis_index("s")
        w = s * NC + c

        dsplat = [jnp.full((16,), d, dtype=jnp.int32) for d in range(D)]

        def fill(out_ref, h):
            @plsc.parallel_loop(0, QH, 1, unroll=8)
            def _body(q):
                cc = q // 8
                c0 = (q % 8) * 16
                xv = xcol_v[pl.ds(h * BH + q * 16, 16)]
                for d in range(D):
                    out_ref[d // 8, cc, d % 8, pl.ds(c0, 16)] = (
                        plsc.load_gather(tbl_v, [dsplat[d], xv])
                    )

        def wait_half(out_ref):
            pltpu.make_async_copy(
                out_hbm.at[0, 0, :, pl.ds(0, BH // 128)], out_ref, sem
            ).wait()

        def run_pair(p, wait_a, wait_b):
            i = p // NJ
            jj = p % NJ
            j = jj + jnp.where(jj >= i, 1, 0).astype(jj.dtype)
            pltpu.sync_copy(tab_hbm.at[i, j], tbl_v)
            pltpu.sync_copy(xt_hbm.at[i], xcol_v)
            if wait_a:
                wait_half(outa_v)
            fill(outa_v, 0)
            if wait_b:
                wait_half(outb_v)
            pltpu.async_copy(
                outa_v, out_hbm.at[i, jj, :, pl.ds(0, BH // 128)], sem
            )
            fill(outb_v, 1)
            pltpu.async_copy(
                outb_v, out_hbm.at[i, jj, :, pl.ds(BH // 128, BH // 128)], sem
            )

        run_pair(w, False, False)

        def body(t, carry):
            p = w + NW * t

            @pl.when(p < PAIRS)
            def _():
                run_pair(p, True, True)

            return carry

        lax.fori_loop(1, P_IT, body, 0)
        wait_half(outa_v)
        wait_half(outb_v)

    return k


_GATHER = _make_kernel()


def kernel(x, tables):
    tt = tables.transpose(0, 1, 3, 2)
    out6 = _GATHER(x.T, tt)
    out = out6.transpose(0, 1, 2, 4, 3, 5).reshape(F, NJ, D, B)
    return out.transpose(0, 3, 1, 2)


# speedup vs baseline: 71.3722x; 1.4228x over previous
import functools

import jax
import jax.numpy as jnp
from jax import lax
from jax.experimental import pallas as pl
from jax.experimental.pallas import tpu as pltpu
from jax.experimental.pallas import tpu_sc as plsc

F = 26
V = 1000
D = 16
B = 4096
NJ = F - 1

NC = 2
NS = 16
NW = NC * NS
PAIRS = F * NJ
P_IT = -(-PAIRS // NW)
BH = B // 2
QH = BH // 16
CH2 = BH // 128


def _make_kernel():
    mesh = plsc.VectorSubcoreMesh(core_axis_name="c", subcore_axis_name="s")

    @functools.partial(
        pl.kernel,
        mesh=mesh,
        out_type=jax.ShapeDtypeStruct((F, NJ, 2, B // 128, 8, 128), jnp.float32),
        compiler_params=pltpu.CompilerParams(
            use_tc_tiling_on_sc=False, needs_layout_passes=False
        ),
        scratch_types=[
            pltpu.VMEM((D, V), jnp.float32),
            pltpu.VMEM((D, V), jnp.float32),
            pltpu.VMEM((B,), jnp.int32),
            pltpu.VMEM((B,), jnp.int32),
            pltpu.VMEM((2, CH2, 8, 128), jnp.float32),
            pltpu.VMEM((2, CH2, 8, 128), jnp.float32),
            pltpu.SemaphoreType.DMA,
            pltpu.SemaphoreType.DMA,
        ],
    )
    def k(xt_hbm, tab_hbm, out_hbm, tbl0_v, tbl1_v, x0_v, x1_v,
          outa_v, outb_v, sem, sem2):
        c = lax.axis_index("c")
        s = lax.axis_index("s")
        w = s * NC + c

        def fill(out_ref, tbl_ref, x_ref, h):
            @plsc.parallel_loop(0, QH, 1, unroll=16)
            def _body(q):
                cc = q // 8
                c0 = (q % 8) * 16
                xv = x_ref[pl.ds(h * BH + q * 16, 16)]
                for d in range(D):
                    out_ref[d // 8, cc, d % 8, pl.ds(c0, 16)] = (
                        plsc.load_gather(tbl_ref.at[d], [xv])
                    )

        def wait_half(out_ref):
            pltpu.make_async_copy(
                out_hbm.at[0, 0, :, pl.ds(0, CH2)], out_ref, sem
            ).wait()

        def load_pair(p, tbl_ref, x_ref):
            i = p // NJ
            jj = p % NJ
            j = jj + jnp.where(jj >= i, 1, 0).astype(jj.dtype)
            pltpu.async_copy(tab_hbm.at[i, j], tbl_ref, sem2)
            pltpu.async_copy(xt_hbm.at[i], x_ref, sem2)

        def wait_pair(tbl_ref, x_ref):
            pltpu.make_async_copy(tab_hbm.at[0, 0], tbl_ref, sem2).wait()
            pltpu.make_async_copy(xt_hbm.at[0], x_ref, sem2).wait()

        def run_pair(p, tbl_ref, x_ref, wait_ab):
            i = p // NJ
            jj = p % NJ
            wait_pair(tbl_ref, x_ref)
            if wait_ab:
                wait_half(outa_v)
            fill(outa_v, tbl_ref, x_ref, 0)
            if wait_ab:
                wait_half(outb_v)
            pltpu.async_copy(
                outa_v, out_hbm.at[i, jj, :, pl.ds(0, CH2)], sem
            )
            fill(outb_v, tbl_ref, x_ref, 1)
            pltpu.async_copy(
                outb_v, out_hbm.at[i, jj, :, pl.ds(CH2, CH2)], sem
            )

        def prefetch(t, tbl_ref, x_ref):
            p = w + NW * t

            @pl.when(p < PAIRS)
            def _():
                load_pair(p, tbl_ref, x_ref)

        load_pair(w, tbl0_v, x0_v)
        prefetch(1, tbl1_v, x1_v)
        run_pair(w, tbl0_v, x0_v, False)

        def body(t2, carry):
            p1 = w + NW * (2 * t2 + 1)
            p2 = w + NW * (2 * t2 + 2)

            @pl.when(p1 < PAIRS)
            def _():
                prefetch(2 * t2 + 2, tbl0_v, x0_v)
                run_pair(p1, tbl1_v, x1_v, True)

            @pl.when(p2 < PAIRS)
            def _():
                prefetch(2 * t2 + 3, tbl1_v, x1_v)
                run_pair(p2, tbl0_v, x0_v, True)

            return carry

        lax.fori_loop(0, (P_IT - 1) // 2, body, 0)
        wait_half(outa_v)
        wait_half(outb_v)

    return k


_GATHER = _make_kernel()


def kernel(x, tables):
    tt = tables.transpose(0, 1, 3, 2)
    out6 = _GATHER(x.T, tt)
    out = out6.transpose(0, 1, 2, 4, 3, 5).reshape(F, NJ, D, B)
    return out.transpose(0, 3, 1, 2)
